# baseline (speedup 1.0000x reference)
import functools

import jax
import jax.numpy as jnp
from jax import lax
from jax.experimental import pallas as pl
from jax.experimental.pallas import tpu as pltpu
from jax.experimental.pallas import tpu_sc as plsc

N, D, E, H, C = 2048, 768, 8, 3072, 1000
A = 2 * N
FB = 256
PADT = A + E * FB
NB = PADT // FB
NC, NS, L = 2, 16, 16


def _layernorm(x, g, b, eps=1e-5):
    m = jnp.mean(x, axis=-1, keepdims=True)
    v = jnp.mean((x - m) ** 2, axis=-1, keepdims=True)
    return (x - m) / jnp.sqrt(v + eps) * g + b


def _bf16_dot(a, b):
    return jax.lax.dot_general(
        a.astype(jnp.bfloat16), b.astype(jnp.bfloat16),
        (((1,), (0,)), ((), ())), preferred_element_type=jnp.float32)


def _router_kernel(x_ref, Win_ref, bin_ref, gin_ref, bim_ref, Wr_ref, br_ref,
                   h_ref, h2_ref, i1_ref, i2_ref, p1_ref, p2_ref):
    h = _bf16_dot(x_ref[...], Win_ref[...]) + bin_ref[...][None, :]
    h = _layernorm(h, gin_ref[...][None, :], bim_ref[...][None, :])
    h_ref[...] = h
    h2_ref[...] = h.astype(jnp.bfloat16)
    logits = _bf16_dot(h, Wr_ref[...]) + br_ref[...][None, :]
    ei = lax.broadcasted_iota(jnp.int32, logits.shape, 1)
    v1 = jnp.max(logits, axis=-1, keepdims=True)
    i1 = jnp.min(jnp.where(logits == v1, ei, E), axis=-1, keepdims=True)
    l2 = jnp.where(ei == i1, -jnp.inf, logits)
    v2 = jnp.max(l2, axis=-1, keepdims=True)
    i2 = jnp.min(jnp.where(l2 == v2, ei, E), axis=-1, keepdims=True)
    p1 = 1.0 / (1.0 + jnp.exp(v2 - v1))
    i1_ref[...] = i1
    i2_ref[...] = i2
    p1_ref[...] = p1
    p2_ref[...] = 1.0 - p1


def _lane_scalar(vec, lane):
    li = lax.broadcasted_iota(jnp.int32, (L,), 0)
    return jnp.sum(jnp.where(li == lane, vec, 0))


def _sc_route_body(i1_hbm, i2_hbm, order_hbm, pos1_hbm, pos2_hbm, be_hbm,
                   nb_hbm, ids_v, posbuf_v, order_v, posall_v, misc_v,
                   cnt_sh, pos_sh):
    c = lax.axis_index("c")
    s = lax.axis_index("s")
    apt = A // NS
    nv = apt // L

    @pl.when(c == 0)
    def _core0():
        @pl.when(s < NS // 2)
        def _():
            pltpu.sync_copy(i1_hbm.at[pl.ds(s * apt, apt)], ids_v)

        @pl.when(s >= NS // 2)
        def _():
            pltpu.sync_copy(i2_hbm.at[pl.ds((s - NS // 2) * apt, apt)], ids_v)

        acc = [jnp.zeros((L,), jnp.int32) for _ in range(E)]
        for j in range(nv):
            idv = ids_v[pl.ds(j * L, L)]
            for e in range(E):
                acc[e] = acc[e] + jnp.where(idv == e, 1, 0)
        li = lax.broadcasted_iota(jnp.int32, (L,), 0)
        cnt_vec = jnp.zeros((L,), jnp.int32)
        for e in range(E):
            cnt_vec = cnt_vec + jnp.where(li == e, jnp.sum(acc[e]), 0)
        misc_v[pl.ds(0, L)] = cnt_vec
        pltpu.sync_copy(misc_v.at[pl.ds(0, L)], cnt_sh.at[pl.ds(s * L, L)])
        plsc.subcore_barrier()

        pltpu.sync_copy(cnt_sh, posall_v.at[pl.ds(0, NS * L)])
        total = jnp.zeros((L,), jnp.int32)
        prefix = jnp.zeros((L,), jnp.int32)
        for w in range(NS):
            cw = posall_v[pl.ds(w * L, L)]
            total = total + cw
            prefix = jnp.where(jnp.full((L,), w, jnp.int32) < s, prefix + cw,
                               prefix)
        padded = (total + (FB - 1)) & (-FB)
        ipad = plsc.cumsum(padded)
        offs = ipad - padded
        base_v = offs + prefix
        bases = [_lane_scalar(base_v, e) for e in range(E)]

        for j in range(nv):
            idv = ids_v[pl.ds(j * L, L)]
            posj = jnp.zeros((L,), jnp.int32)
            for e in range(E):
                m = idv == e
                mi = jnp.where(m, 1, 0)
                incl = plsc.cumsum(mi)
                posj = jnp.where(m, bases[e] + incl - mi, posj)
                bases[e] = bases[e] + jnp.sum(mi)
            posbuf_v[pl.ds(j * L, L)] = posj
        pltpu.sync_copy(posbuf_v, pos_sh.at[pl.ds(s * apt, apt)])

        @pl.when(s < NS // 2)
        def _():
            pltpu.sync_copy(posbuf_v, pos1_hbm.at[pl.ds(s * apt, apt)])

        @pl.when(s >= NS // 2)
        def _():
            pltpu.sync_copy(posbuf_v,
                            pos2_hbm.at[pl.ds((s - NS // 2) * apt, apt)])

        plsc.subcore_barrier()

        @pl.when(s == 0)
        def _tile0():
            for k in range(PADT // L):
                order_v[pl.ds(k * L, L)] = jnp.zeros((L,), jnp.int32)
            pltpu.sync_copy(pos_sh, posall_v)
            for k in range(A // L):
                pv = posall_v[pl.ds(k * L, L)]
                tokbase = k * L if k < N // L else k * L - N
                tok = lax.broadcasted_iota(jnp.int32, (L,), 0) + tokbase
                plsc.store_scatter(order_v, [pv], tok)
            pltpu.sync_copy(order_v, order_hbm)

            blkend = ipad >> 8
            bes = [_lane_scalar(blkend, e) for e in range(E)]
            nblk = bes[E - 1]
            for half in range(2):
                bi = lax.broadcasted_iota(jnp.int32, (L,), 0) + half * L
                ex = jnp.zeros((L,), jnp.int32)
                for e in range(E):
                    ex = ex + jnp.where(bi >= bes[e], 1, 0)
                ex = jnp.minimum(ex, E - 1)
                misc_v[pl.ds(half * L, L)] = ex
            pltpu.sync_copy(misc_v.at[pl.ds(0, 2 * L)], be_hbm)
            misc_v[pl.ds(2 * L, L)] = jnp.where(li == 0, nblk, 0)
            pltpu.sync_copy(misc_v.at[pl.ds(2 * L, 8)], nb_hbm)


def _ffn_kernel(be_ref, nb_ref, ord_ref, hb_ref, W1_ref, b1_ref, W2_ref,
                b2_ref, ys_ref, W1c_ref, W2c_ref, last_ref):
    b = pl.program_id(0)

    @pl.when(b < nb_ref[0])
    def _():
        @pl.when((b == 0) | (be_ref[b] != last_ref[0]))
        def _recast():
            W1c_ref[...] = W1_ref[0].astype(jnp.bfloat16)
            W2c_ref[...] = W2_ref[0].astype(jnp.bfloat16)
            last_ref[0] = be_ref[b]

        ids = ord_ref[0, 0][:, None]
        ci = lax.broadcasted_iota(jnp.int32, (FB, N), 1)
        onehot = (ci == ids).astype(jnp.bfloat16)
        sel = jax.lax.dot_general(onehot, hb_ref[...],
                                  (((1,), (0,)), ((), ())),
                                  preferred_element_type=jnp.float32)
        u = jax.lax.dot_general(sel.astype(jnp.bfloat16), W1c_ref[...],
                                (((1,), (0,)), ((), ())),
                                preferred_element_type=jnp.float32)
        u = jax.nn.gelu(u + b1_ref[0, 0][None, :])
        o = jax.lax.dot_general(u.astype(jnp.bfloat16), W2c_ref[...],
                                (((1,), (0,)), ((), ())),
                                preferred_element_type=jnp.float32)
        ys_ref[...] = o + b2_ref[0, 0][None, :]


def _sc_combine_body(ys_hbm, pos1_hbm, pos2_hbm, p1_hbm, p2_hbm, mixed_hbm,
                     i1_v, i2_v, p1_v, p2_v, rows1_v, rows2_v, sem):
    wid = lax.axis_index("s") * NC + lax.axis_index("c")
    tpt = N // (NC * NS)
    base = wid * tpt
    pltpu.sync_copy(pos1_hbm.at[pl.ds(base, tpt)], i1_v)
    pltpu.sync_copy(pos2_hbm.at[pl.ds(base, tpt)], i2_v)
    pltpu.sync_copy(p1_hbm.at[pl.ds(base, tpt)], p1_v)
    pltpu.sync_copy(p2_hbm.at[pl.ds(base, tpt)], p2_v)
    pltpu.async_copy(ys_hbm.at[i1_v], rows1_v, sem).wait()
    pltpu.async_copy(ys_hbm.at[i2_v], rows2_v, sem).wait()

    ps1, ps2 = [], []
    for g in range(tpt // L):
        p1v = p1_v[pl.ds(g * L, L)]
        p2v = p2_v[pl.ds(g * L, L)]
        for tt in range(L):
            ps1.append(p1v[tt])
            ps2.append(p2v[tt])

    def col_body(k, carry):
        sl = pl.ds(k * L, L)
        for t in range(tpt):
            rows1_v[t, sl] = rows1_v[t, sl] * ps1[t] + rows2_v[t, sl] * ps2[t]
        return carry

    lax.fori_loop(0, D // L, col_body, 0)
    pltpu.sync_copy(rows1_v, mixed_hbm.at[pl.ds(base, tpt)])


def _final_kernel(h_ref, acc_ref, gmo_ref, bmo_ref, gou_ref, bou_ref,
                  Wc_ref, bc_ref, out_ref):
    moe = _layernorm(h_ref[...] + acc_ref[...], gmo_ref[...][None, :],
                     bmo_ref[...][None, :])
    final = _layernorm(moe, gou_ref[...][None, :], bou_ref[...][None, :])
    out_ref[...] = _bf16_dot(final, Wc_ref[...]) + bc_ref[...][None, :]


def kernel(x, Win, bin_, g_in, b_in, Wr, br, W1, b1, W2, b2, g_moe, b_moe,
           g_out, b_out, Wc, bc):
    f32 = jnp.float32
    i32 = jnp.int32

    h, h2, i1, i2, p1, p2 = pl.pallas_call(
        _router_kernel,
        out_shape=(jax.ShapeDtypeStruct((N, D), f32),
                   jax.ShapeDtypeStruct((N, D), jnp.bfloat16),
                   jax.ShapeDtypeStruct((N, 1), i32),
                   jax.ShapeDtypeStruct((N, 1), i32),
                   jax.ShapeDtypeStruct((N, 1), f32),
                   jax.ShapeDtypeStruct((N, 1), f32)),
    )(x, Win, bin_, g_in, b_in, Wr, br)
    i1f = i1.reshape(N)
    i2f = i2.reshape(N)
    p1f = p1.reshape(N)
    p2f = p2.reshape(N)

    mesh = plsc.VectorSubcoreMesh(core_axis_name="c", subcore_axis_name="s",
                                  num_cores=NC, num_subcores=NS)
    sc_params = pltpu.CompilerParams(needs_layout_passes=False)

    order, pos1, pos2, be, nb = pl.kernel(
        _sc_route_body,
        compiler_params=sc_params,
        out_type=(jax.ShapeDtypeStruct((PADT,), i32),
                  jax.ShapeDtypeStruct((N,), i32),
                  jax.ShapeDtypeStruct((N,), i32),
                  jax.ShapeDtypeStruct((2 * L,), i32),
                  jax.ShapeDtypeStruct((8,), i32)),
        mesh=mesh,
        scratch_types=[
            pltpu.VMEM((A // NS,), i32),
            pltpu.VMEM((A // NS,), i32),
            pltpu.VMEM((PADT,), i32),
            pltpu.VMEM((A,), i32),
            pltpu.VMEM((3 * L,), i32),
            pltpu.VMEM_SHARED((NS * L,), i32),
            pltpu.VMEM_SHARED((A,), i32),
        ],
    )(i1f, i2f)

    b1r = b1.reshape(E, 1, H)
    b2r = b2.reshape(E, 1, D)
    ordr = order.reshape(NB, 1, FB)
    ys = pl.pallas_call(
        _ffn_kernel,
        grid_spec=pltpu.PrefetchScalarGridSpec(
            num_scalar_prefetch=2,
            grid=(NB,),
            in_specs=[
                pl.BlockSpec((1, 1, FB), lambda b, be, nb: (b, 0, 0)),
                pl.BlockSpec((N, D), lambda b, be, nb: (0, 0)),
                pl.BlockSpec((1, D, H), lambda b, be, nb: (be[b], 0, 0)),
                pl.BlockSpec((1, 1, H), lambda b, be, nb: (be[b], 0, 0)),
                pl.BlockSpec((1, H, D), lambda b, be, nb: (be[b], 0, 0)),
                pl.BlockSpec((1, 1, D), lambda b, be, nb: (be[b], 0, 0)),
            ],
            out_specs=pl.BlockSpec((FB, D), lambda b, be, nb: (b, 0)),
            scratch_shapes=[
                pltpu.VMEM((D, H), jnp.bfloat16),
                pltpu.VMEM((H, D), jnp.bfloat16),
                pltpu.SMEM((1,), i32),
            ],
        ),
        out_shape=jax.ShapeDtypeStruct((PADT, D), f32),
    )(be, nb, ordr, h2, W1, b1r, W2, b2r)

    mixed = pl.kernel(
        _sc_combine_body,
        compiler_params=sc_params,
        out_type=jax.ShapeDtypeStruct((N, D), f32),
        mesh=mesh,
        scratch_types=[
            pltpu.VMEM((N // (NC * NS),), i32),
            pltpu.VMEM((N // (NC * NS),), i32),
            pltpu.VMEM((N // (NC * NS),), f32),
            pltpu.VMEM((N // (NC * NS),), f32),
            pltpu.VMEM((N // (NC * NS), D), f32),
            pltpu.VMEM((N // (NC * NS), D), f32),
            pltpu.SemaphoreType.DMA,
        ],
    )(ys, pos1, pos2, p1f, p2f)

    TB = 512
    nt = N // TB
    out = pl.pallas_call(
        _final_kernel,
        grid=(nt,),
        in_specs=[
            pl.BlockSpec((TB, D), lambda t: (t, 0)),
            pl.BlockSpec((TB, D), lambda t: (t, 0)),
            pl.BlockSpec((D,), lambda t: (0,)),
            pl.BlockSpec((D,), lambda t: (0,)),
            pl.BlockSpec((D,), lambda t: (0,)),
            pl.BlockSpec((D,), lambda t: (0,)),
            pl.BlockSpec((D, C), lambda t: (0, 0)),
            pl.BlockSpec((C,), lambda t: (0,)),
        ],
        out_specs=pl.BlockSpec((TB, C), lambda t: (t, 0)),
        out_shape=jax.ShapeDtypeStruct((N, C), f32),
    )(h, mixed, g_moe, b_moe, g_out, b_out, Wc, bc)
    return out


# speedup vs baseline: 1.0249x; 1.0249x over previous
import functools

import jax
import jax.numpy as jnp
from jax import lax
from jax.experimental import pallas as pl
from jax.experimental.pallas import tpu as pltpu
from jax.experimental.pallas import tpu_sc as plsc

N, D, E, H, C = 2048, 768, 8, 3072, 1000
A = 2 * N
FB = 256
PADT = A + E * FB
NB = PADT // FB
NC, NS, L = 2, 16, 16


def _layernorm(x, g, b, eps=1e-5):
    m = jnp.mean(x, axis=-1, keepdims=True)
    v = jnp.mean((x - m) ** 2, axis=-1, keepdims=True)
    return (x - m) / jnp.sqrt(v + eps) * g + b


def _bf16_dot(a, b):
    return jax.lax.dot_general(
        a.astype(jnp.bfloat16), b.astype(jnp.bfloat16),
        (((1,), (0,)), ((), ())), preferred_element_type=jnp.float32)


def _router_kernel(x_ref, Win_ref, bin_ref, gin_ref, bim_ref, Wr_ref, br_ref,
                   h_ref, h2_ref, i1_ref, i2_ref, p1_ref, p2_ref):
    h = _bf16_dot(x_ref[...], Win_ref[...]) + bin_ref[...][None, :]
    h = _layernorm(h, gin_ref[...][None, :], bim_ref[...][None, :])
    h_ref[...] = h
    h2_ref[...] = h.astype(jnp.bfloat16)
    logits = _bf16_dot(h, Wr_ref[...]) + br_ref[...][None, :]
    ei = lax.broadcasted_iota(jnp.int32, logits.shape, 1)
    v1 = jnp.max(logits, axis=-1, keepdims=True)
    i1 = jnp.min(jnp.where(logits == v1, ei, E), axis=-1, keepdims=True)
    l2 = jnp.where(ei == i1, -jnp.inf, logits)
    v2 = jnp.max(l2, axis=-1, keepdims=True)
    i2 = jnp.min(jnp.where(l2 == v2, ei, E), axis=-1, keepdims=True)
    p1 = 1.0 / (1.0 + jnp.exp(v2 - v1))
    i1_ref[...] = i1
    i2_ref[...] = i2
    p1_ref[...] = p1
    p2_ref[...] = 1.0 - p1


def _lane_scalar(vec, lane):
    li = lax.broadcasted_iota(jnp.int32, (L,), 0)
    return jnp.sum(jnp.where(li == lane, vec, 0))


def _sc_route_body(i1_hbm, i2_hbm, order_hbm, pos1_hbm, pos2_hbm, be_hbm,
                   nb_hbm, ids_v, posbuf_v, order_v, posall_v, misc_v,
                   cnt_sh, pos_sh):
    c = lax.axis_index("c")
    s = lax.axis_index("s")
    apt = A // NS
    nv = apt // L

    @pl.when(c == 0)
    def _core0():
        @pl.when(s < NS // 2)
        def _():
            pltpu.sync_copy(i1_hbm.at[pl.ds(s * apt, apt)], ids_v)

        @pl.when(s >= NS // 2)
        def _():
            pltpu.sync_copy(i2_hbm.at[pl.ds((s - NS // 2) * apt, apt)], ids_v)

        acc = [jnp.zeros((L,), jnp.int32) for _ in range(E)]
        for j in range(nv):
            idv = ids_v[pl.ds(j * L, L)]
            for e in range(E):
                acc[e] = acc[e] + jnp.where(idv == e, 1, 0)
        li = lax.broadcasted_iota(jnp.int32, (L,), 0)
        cnt_vec = jnp.zeros((L,), jnp.int32)
        for e in range(E):
            cnt_vec = cnt_vec + jnp.where(li == e, jnp.sum(acc[e]), 0)
        misc_v[pl.ds(0, L)] = cnt_vec
        pltpu.sync_copy(misc_v.at[pl.ds(0, L)], cnt_sh.at[pl.ds(s * L, L)])
        plsc.subcore_barrier()

        pltpu.sync_copy(cnt_sh, posall_v.at[pl.ds(0, NS * L)])
        total = jnp.zeros((L,), jnp.int32)
        prefix = jnp.zeros((L,), jnp.int32)
        for w in range(NS):
            cw = posall_v[pl.ds(w * L, L)]
            total = total + cw
            prefix = jnp.where(jnp.full((L,), w, jnp.int32) < s, prefix + cw,
                               prefix)
        padded = (total + (FB - 1)) & (-FB)
        ipad = plsc.cumsum(padded)
        offs = ipad - padded
        base_v = offs + prefix
        bases = [_lane_scalar(base_v, e) for e in range(E)]

        for j in range(nv):
            idv = ids_v[pl.ds(j * L, L)]
            posj = jnp.zeros((L,), jnp.int32)
            for e in range(E):
                m = idv == e
                mi = jnp.where(m, 1, 0)
                incl = plsc.cumsum(mi)
                posj = jnp.where(m, bases[e] + incl - mi, posj)
                bases[e] = bases[e] + jnp.sum(mi)
            posbuf_v[pl.ds(j * L, L)] = posj
        pltpu.sync_copy(posbuf_v, pos_sh.at[pl.ds(s * apt, apt)])

        @pl.when(s < NS // 2)
        def _():
            pltpu.sync_copy(posbuf_v, pos1_hbm.at[pl.ds(s * apt, apt)])

        @pl.when(s >= NS // 2)
        def _():
            pltpu.sync_copy(posbuf_v,
                            pos2_hbm.at[pl.ds((s - NS // 2) * apt, apt)])

        plsc.subcore_barrier()

        @pl.when(s == 0)
        def _tile0():
            for k in range(PADT // L):
                order_v[pl.ds(k * L, L)] = jnp.zeros((L,), jnp.int32)
            pltpu.sync_copy(pos_sh, posall_v)
            for k in range(A // L):
                pv = posall_v[pl.ds(k * L, L)]
                tokbase = k * L if k < N // L else k * L - N
                tok = lax.broadcasted_iota(jnp.int32, (L,), 0) + tokbase
                plsc.store_scatter(order_v, [pv], tok)
            pltpu.sync_copy(order_v, order_hbm)

            blkend = ipad >> 8
            bes = [_lane_scalar(blkend, e) for e in range(E)]
            nblk = bes[E - 1]
            for half in range(2):
                bi = lax.broadcasted_iota(jnp.int32, (L,), 0) + half * L
                ex = jnp.zeros((L,), jnp.int32)
                for e in range(E):
                    ex = ex + jnp.where(bi >= bes[e], 1, 0)
                ex = jnp.minimum(ex, E - 1)
                misc_v[pl.ds(half * L, L)] = ex
            pltpu.sync_copy(misc_v.at[pl.ds(0, 2 * L)], be_hbm)
            misc_v[pl.ds(2 * L, L)] = jnp.where(li == 0, nblk, 0)
            pltpu.sync_copy(misc_v.at[pl.ds(2 * L, 8)], nb_hbm)


def _ffn_kernel(be_ref, nb_ref, ord_ref, hb_ref, W1_ref, b1_ref, W2_ref,
                b2_ref, ys_ref):
    @pl.when(pl.program_id(0) < nb_ref[0])
    def _():
        ids = ord_ref[0, 0][:, None]
        ci = lax.broadcasted_iota(jnp.int32, (FB, N), 1)
        onehot = (ci == ids).astype(jnp.bfloat16)
        sel = jax.lax.dot_general(onehot, hb_ref[...],
                                  (((1,), (0,)), ((), ())),
                                  preferred_element_type=jnp.float32)
        u = _bf16_dot(sel, W1_ref[0]) + b1_ref[0, 0][None, :]
        u = jax.nn.gelu(u)
        ys_ref[...] = _bf16_dot(u, W2_ref[0]) + b2_ref[0, 0][None, :]


def _sc_combine_body(ys_hbm, pos1_hbm, pos2_hbm, p1_hbm, p2_hbm, mixed_hbm,
                     i1_v, i2_v, p1_v, p2_v, rows1_v, rows2_v, sem):
    wid = lax.axis_index("s") * NC + lax.axis_index("c")
    tpt = N // (NC * NS)
    base = wid * tpt
    pltpu.sync_copy(pos1_hbm.at[pl.ds(base, tpt)], i1_v)
    pltpu.sync_copy(pos2_hbm.at[pl.ds(base, tpt)], i2_v)
    pltpu.sync_copy(p1_hbm.at[pl.ds(base, tpt)], p1_v)
    pltpu.sync_copy(p2_hbm.at[pl.ds(base, tpt)], p2_v)
    pltpu.async_copy(ys_hbm.at[i1_v], rows1_v, sem).wait()
    pltpu.async_copy(ys_hbm.at[i2_v], rows2_v, sem).wait()

    ps1, ps2 = [], []
    for g in range(tpt // L):
        p1v = p1_v[pl.ds(g * L, L)]
        p2v = p2_v[pl.ds(g * L, L)]
        for tt in range(L):
            ps1.append(p1v[tt])
            ps2.append(p2v[tt])

    def col_body(k, carry):
        sl = pl.ds(k * L, L)
        for t in range(tpt):
            rows1_v[t, sl] = rows1_v[t, sl] * ps1[t] + rows2_v[t, sl] * ps2[t]
        return carry

    lax.fori_loop(0, D // L, col_body, 0)
    pltpu.sync_copy(rows1_v, mixed_hbm.at[pl.ds(base, tpt)])


def _final_kernel(h_ref, acc_ref, gmo_ref, bmo_ref, gou_ref, bou_ref,
                  Wc_ref, bc_ref, out_ref):
    moe = _layernorm(h_ref[...] + acc_ref[...], gmo_ref[...][None, :],
                     bmo_ref[...][None, :])
    final = _layernorm(moe, gou_ref[...][None, :], bou_ref[...][None, :])
    out_ref[...] = _bf16_dot(final, Wc_ref[...]) + bc_ref[...][None, :]


def kernel(x, Win, bin_, g_in, b_in, Wr, br, W1, b1, W2, b2, g_moe, b_moe,
           g_out, b_out, Wc, bc):
    f32 = jnp.float32
    i32 = jnp.int32

    h, h2, i1, i2, p1, p2 = pl.pallas_call(
        _router_kernel,
        out_shape=(jax.ShapeDtypeStruct((N, D), f32),
                   jax.ShapeDtypeStruct((N, D), jnp.bfloat16),
                   jax.ShapeDtypeStruct((N, 1), i32),
                   jax.ShapeDtypeStruct((N, 1), i32),
                   jax.ShapeDtypeStruct((N, 1), f32),
                   jax.ShapeDtypeStruct((N, 1), f32)),
    )(x, Win, bin_, g_in, b_in, Wr, br)
    i1f = i1.reshape(N)
    i2f = i2.reshape(N)
    p1f = p1.reshape(N)
    p2f = p2.reshape(N)

    mesh = plsc.VectorSubcoreMesh(core_axis_name="c", subcore_axis_name="s",
                                  num_cores=NC, num_subcores=NS)
    sc_params = pltpu.CompilerParams(needs_layout_passes=False)

    order, pos1, pos2, be, nb = pl.kernel(
        _sc_route_body,
        compiler_params=sc_params,
        out_type=(jax.ShapeDtypeStruct((PADT,), i32),
                  jax.ShapeDtypeStruct((N,), i32),
                  jax.ShapeDtypeStruct((N,), i32),
                  jax.ShapeDtypeStruct((2 * L,), i32),
                  jax.ShapeDtypeStruct((8,), i32)),
        mesh=mesh,
        scratch_types=[
            pltpu.VMEM((A // NS,), i32),
            pltpu.VMEM((A // NS,), i32),
            pltpu.VMEM((PADT,), i32),
            pltpu.VMEM((A,), i32),
            pltpu.VMEM((3 * L,), i32),
            pltpu.VMEM_SHARED((NS * L,), i32),
            pltpu.VMEM_SHARED((A,), i32),
        ],
    )(i1f, i2f)

    b1r = b1.reshape(E, 1, H)
    b2r = b2.reshape(E, 1, D)
    ordr = order.reshape(NB, 1, FB)
    ys = pl.pallas_call(
        _ffn_kernel,
        grid_spec=pltpu.PrefetchScalarGridSpec(
            num_scalar_prefetch=2,
            grid=(NB,),
            in_specs=[
                pl.BlockSpec((1, 1, FB), lambda b, be, nb: (b, 0, 0)),
                pl.BlockSpec((N, D), lambda b, be, nb: (0, 0)),
                pl.BlockSpec((1, D, H), lambda b, be, nb: (be[b], 0, 0)),
                pl.BlockSpec((1, 1, H), lambda b, be, nb: (be[b], 0, 0)),
                pl.BlockSpec((1, H, D), lambda b, be, nb: (be[b], 0, 0)),
                pl.BlockSpec((1, 1, D), lambda b, be, nb: (be[b], 0, 0)),
            ],
            out_specs=pl.BlockSpec((FB, D), lambda b, be, nb: (b, 0)),
        ),
        out_shape=jax.ShapeDtypeStruct((PADT, D), f32),
    )(be, nb, ordr, h2, W1, b1r, W2, b2r)

    mixed = pl.kernel(
        _sc_combine_body,
        compiler_params=sc_params,
        out_type=jax.ShapeDtypeStruct((N, D), f32),
        mesh=mesh,
        scratch_types=[
            pltpu.VMEM((N // (NC * NS),), i32),
            pltpu.VMEM((N // (NC * NS),), i32),
            pltpu.VMEM((N // (NC * NS),), f32),
            pltpu.VMEM((N // (NC * NS),), f32),
            pltpu.VMEM((N // (NC * NS), D), f32),
            pltpu.VMEM((N // (NC * NS), D), f32),
            pltpu.SemaphoreType.DMA,
        ],
    )(ys, pos1, pos2, p1f, p2f)

    TB = 512
    nt = N // TB
    out = pl.pallas_call(
        _final_kernel,
        grid=(nt,),
        in_specs=[
            pl.BlockSpec((TB, D), lambda t: (t, 0)),
            pl.BlockSpec((TB, D), lambda t: (t, 0)),
            pl.BlockSpec((D,), lambda t: (0,)),
            pl.BlockSpec((D,), lambda t: (0,)),
            pl.BlockSpec((D,), lambda t: (0,)),
            pl.BlockSpec((D,), lambda t: (0,)),
            pl.BlockSpec((D, C), lambda t: (0, 0)),
            pl.BlockSpec((C,), lambda t: (0,)),
        ],
        out_specs=pl.BlockSpec((TB, C), lambda t: (t, 0)),
        out_shape=jax.ShapeDtypeStruct((N, C), f32),
    )(h, mixed, g_moe, b_moe, g_out, b_out, Wc, bc)
    return out


# speedup vs baseline: 1.0276x; 1.0027x over previous
import jax
import jax.numpy as jnp
from jax import lax
from jax.experimental import pallas as pl
from jax.experimental.pallas import tpu as pltpu
from jax.experimental.pallas import tpu_sc as plsc

N, D, E, H, C = 2048, 768, 8, 3072, 1000
A = 2 * N
FB = 256
PADT = A + E * FB
NB = PADT // FB
NC, NS, L = 2, 16, 16


def _layernorm(x, g, b, eps=1e-5):
    m = jnp.mean(x, axis=-1, keepdims=True)
    v = jnp.mean((x - m) ** 2, axis=-1, keepdims=True)
    return (x - m) / jnp.sqrt(v + eps) * g + b


def _bf16_dot(a, b):
    return jax.lax.dot_general(
        a.astype(jnp.bfloat16), b.astype(jnp.bfloat16),
        (((1,), (0,)), ((), ())), preferred_element_type=jnp.float32)


def _router_kernel(x_ref, Win_ref, bin_ref, gin_ref, bim_ref, Wr_ref, br_ref,
                   h_ref, h2_ref, i1_ref, i2_ref, p1_ref, p2_ref):
    h = _bf16_dot(x_ref[...], Win_ref[...]) + bin_ref[...][None, :]
    h = _layernorm(h, gin_ref[...][None, :], bim_ref[...][None, :])
    h_ref[...] = h
    h2_ref[...] = h.astype(jnp.bfloat16)
    logits = _bf16_dot(h, Wr_ref[...]) + br_ref[...][None, :]
    ei = lax.broadcasted_iota(jnp.int32, logits.shape, 1)
    v1 = jnp.max(logits, axis=-1, keepdims=True)
    i1 = jnp.min(jnp.where(logits == v1, ei, E), axis=-1, keepdims=True)
    l2 = jnp.where(ei == i1, -jnp.inf, logits)
    v2 = jnp.max(l2, axis=-1, keepdims=True)
    i2 = jnp.min(jnp.where(l2 == v2, ei, E), axis=-1, keepdims=True)
    p1 = 1.0 / (1.0 + jnp.exp(v2 - v1))
    i1_ref[...] = i1
    i2_ref[...] = i2
    p1_ref[...] = p1
    p2_ref[...] = 1.0 - p1


def _lane_scalar(vec, lane):
    li = lax.broadcasted_iota(jnp.int32, (L,), 0)
    return jnp.sum(jnp.where(li == lane, vec, 0))


def _sc_route_body(i1_hbm, i2_hbm, order_hbm, pos1_hbm, pos2_hbm, be_hbm,
                   nb_hbm, ids_v, posbuf_v, order_v, posall_v, misc_v,
                   cnt_sh, pos_sh):
    c = lax.axis_index("c")
    s = lax.axis_index("s")
    apt = A // NS
    nv = apt // L

    @pl.when(c == 0)
    def _core0():
        @pl.when(s < NS // 2)
        def _():
            pltpu.sync_copy(i1_hbm.at[pl.ds(s * apt, apt)], ids_v)

        @pl.when(s >= NS // 2)
        def _():
            pltpu.sync_copy(i2_hbm.at[pl.ds((s - NS // 2) * apt, apt)], ids_v)

        acc = [jnp.zeros((L,), jnp.int32) for _ in range(E)]
        for j in range(nv):
            idv = ids_v[pl.ds(j * L, L)]
            for e in range(E):
                acc[e] = acc[e] + jnp.where(idv == e, 1, 0)
        li = lax.broadcasted_iota(jnp.int32, (L,), 0)
        cnt_vec = jnp.zeros((L,), jnp.int32)
        for e in range(E):
            cnt_vec = cnt_vec + jnp.where(li == e, jnp.sum(acc[e]), 0)
        misc_v[pl.ds(0, L)] = cnt_vec
        pltpu.sync_copy(misc_v.at[pl.ds(0, L)], cnt_sh.at[pl.ds(s * L, L)])
        plsc.subcore_barrier()

        pltpu.sync_copy(cnt_sh, posall_v.at[pl.ds(0, NS * L)])
        total = jnp.zeros((L,), jnp.int32)
        prefix = jnp.zeros((L,), jnp.int32)
        for w in range(NS):
            cw = posall_v[pl.ds(w * L, L)]
            total = total + cw
            prefix = jnp.where(jnp.full((L,), w, jnp.int32) < s, prefix + cw,
                               prefix)
        padded = (total + (FB - 1)) & (-FB)
        ipad = plsc.cumsum(padded)
        offs = ipad - padded
        base_v = offs + prefix
        bases = [_lane_scalar(base_v, e) for e in range(E)]

        for j in range(nv):
            idv = ids_v[pl.ds(j * L, L)]
            posj = jnp.zeros((L,), jnp.int32)
            for e in range(E):
                m = idv == e
                mi = jnp.where(m, 1, 0)
                incl = plsc.cumsum(mi)
                posj = jnp.where(m, bases[e] + incl - mi, posj)
                bases[e] = bases[e] + jnp.sum(mi)
            posbuf_v[pl.ds(j * L, L)] = posj
        pltpu.sync_copy(posbuf_v, pos_sh.at[pl.ds(s * apt, apt)])

        @pl.when(s < NS // 2)
        def _():
            pltpu.sync_copy(posbuf_v, pos1_hbm.at[pl.ds(s * apt, apt)])

        @pl.when(s >= NS // 2)
        def _():
            pltpu.sync_copy(posbuf_v,
                            pos2_hbm.at[pl.ds((s - NS // 2) * apt, apt)])

        plsc.subcore_barrier()

        @pl.when(s == 0)
        def _tile0():
            for k in range(PADT // L):
                order_v[pl.ds(k * L, L)] = jnp.zeros((L,), jnp.int32)
            pltpu.sync_copy(pos_sh, posall_v)
            for k in range(A // L):
                pv = posall_v[pl.ds(k * L, L)]
                tokbase = k * L if k < N // L else k * L - N
                tok = lax.broadcasted_iota(jnp.int32, (L,), 0) + tokbase
                plsc.store_scatter(order_v, [pv], tok)
            pltpu.sync_copy(order_v, order_hbm)

            blkend = ipad >> 8
            bes = [_lane_scalar(blkend, e) for e in range(E)]
            nblk = bes[E - 1]
            for half in range(2):
                bi = lax.broadcasted_iota(jnp.int32, (L,), 0) + half * L
                ex = jnp.zeros((L,), jnp.int32)
                for e in range(E):
                    ex = ex + jnp.where(bi >= bes[e], 1, 0)
                ex = jnp.minimum(ex, E - 1)
                misc_v[pl.ds(half * L, L)] = ex
            pltpu.sync_copy(misc_v.at[pl.ds(0, 2 * L)], be_hbm)
            misc_v[pl.ds(2 * L, L)] = jnp.where(li == 0, nblk, 0)
            pltpu.sync_copy(misc_v.at[pl.ds(2 * L, 8)], nb_hbm)


def _ffn_kernel(be_ref, nb_ref, ord_ref, hb_ref, W1_ref, b1_ref, W2_ref,
                b2_ref, ys_ref):
    @pl.when(pl.program_id(0) < nb_ref[0])
    def _():
        ids = ord_ref[0, 0][:, None]
        ci = lax.broadcasted_iota(jnp.int32, (FB, N), 1)
        onehot = (ci == ids).astype(jnp.bfloat16)
        sel = jax.lax.dot_general(onehot, hb_ref[...],
                                  (((1,), (0,)), ((), ())),
                                  preferred_element_type=jnp.float32)
        u = _bf16_dot(sel, W1_ref[0]) + b1_ref[0, 0][None, :]
        u = jax.nn.gelu(u)
        ys_ref[...] = _bf16_dot(u, W2_ref[0]) + b2_ref[0, 0][None, :]


def _sc_combine_body(ys_hbm, pos1_hbm, pos2_hbm, p1_hbm, p2_hbm, mixed_hbm,
                     i1_v, i2_v, p1_v, p2_v, rows1_v, rows2_v, sem):
    wid = lax.axis_index("s") * NC + lax.axis_index("c")
    tpt = N // (NC * NS)
    base = wid * tpt
    pltpu.sync_copy(pos1_hbm.at[pl.ds(base, tpt)], i1_v)
    pltpu.sync_copy(pos2_hbm.at[pl.ds(base, tpt)], i2_v)
    pltpu.sync_copy(p1_hbm.at[pl.ds(base, tpt)], p1_v)
    pltpu.sync_copy(p2_hbm.at[pl.ds(base, tpt)], p2_v)
    pltpu.async_copy(ys_hbm.at[i1_v], rows1_v, sem).wait()
    pltpu.async_copy(ys_hbm.at[i2_v], rows2_v, sem).wait()

    ps1, ps2 = [], []
    for g in range(tpt // L):
        p1v = p1_v[pl.ds(g * L, L)]
        p2v = p2_v[pl.ds(g * L, L)]
        for tt in range(L):
            ps1.append(p1v[tt])
            ps2.append(p2v[tt])

    def col_body(k, carry):
        sl = pl.ds(k * L, L)
        for t in range(tpt):
            rows1_v[t, sl] = rows1_v[t, sl] * ps1[t] + rows2_v[t, sl] * ps2[t]
        return carry

    lax.fori_loop(0, D // L, col_body, 0)
    pltpu.sync_copy(rows1_v, mixed_hbm.at[pl.ds(base, tpt)])


def _final_kernel(h_ref, acc_ref, gmo_ref, bmo_ref, gou_ref, bou_ref,
                  Wc_ref, bc_ref, out_ref):
    moe = _layernorm(h_ref[...] + acc_ref[...], gmo_ref[...][None, :],
                     bmo_ref[...][None, :])
    final = _layernorm(moe, gou_ref[...][None, :], bou_ref[...][None, :])
    out_ref[...] = _bf16_dot(final, Wc_ref[...]) + bc_ref[...][None, :]


def kernel(x, Win, bin_, g_in, b_in, Wr, br, W1, b1, W2, b2, g_moe, b_moe,
           g_out, b_out, Wc, bc):
    f32 = jnp.float32
    i32 = jnp.int32

    h, h2, i1, i2, p1, p2 = pl.pallas_call(
        _router_kernel,
        out_shape=(jax.ShapeDtypeStruct((N, D), f32),
                   jax.ShapeDtypeStruct((N, D), jnp.bfloat16),
                   jax.ShapeDtypeStruct((N, 1), i32),
                   jax.ShapeDtypeStruct((N, 1), i32),
                   jax.ShapeDtypeStruct((N, 1), f32),
                   jax.ShapeDtypeStruct((N, 1), f32)),
    )(x, Win, bin_, g_in, b_in, Wr, br)
    i1f = i1.reshape(N)
    i2f = i2.reshape(N)
    p1f = p1.reshape(N)
    p2f = p2.reshape(N)

    mesh = plsc.VectorSubcoreMesh(core_axis_name="c", subcore_axis_name="s",
                                  num_cores=NC, num_subcores=NS)
    sc_params = pltpu.CompilerParams(needs_layout_passes=False)

    order, pos1, pos2, be, nb = pl.kernel(
        _sc_route_body,
        compiler_params=sc_params,
        out_type=(jax.ShapeDtypeStruct((PADT,), i32),
                  jax.ShapeDtypeStruct((N,), i32),
                  jax.ShapeDtypeStruct((N,), i32),
                  jax.ShapeDtypeStruct((2 * L,), i32),
                  jax.ShapeDtypeStruct((8,), i32)),
        mesh=mesh,
        scratch_types=[
            pltpu.VMEM((A // NS,), i32),
            pltpu.VMEM((A // NS,), i32),
            pltpu.VMEM((PADT,), i32),
            pltpu.VMEM((A,), i32),
            pltpu.VMEM((3 * L,), i32),
            pltpu.VMEM_SHARED((NS * L,), i32),
            pltpu.VMEM_SHARED((A,), i32),
        ],
    )(i1f, i2f)

    b1r = b1.reshape(E, 1, H)
    b2r = b2.reshape(E, 1, D)
    ordr = order.reshape(NB, 1, FB)
    ys = pl.pallas_call(
        _ffn_kernel,
        grid_spec=pltpu.PrefetchScalarGridSpec(
            num_scalar_prefetch=2,
            grid=(NB,),
            in_specs=[
                pl.BlockSpec((1, 1, FB), lambda b, be, nb: (b, 0, 0)),
                pl.BlockSpec((N, D), lambda b, be, nb: (0, 0)),
                pl.BlockSpec((1, D, H), lambda b, be, nb: (be[b], 0, 0)),
                pl.BlockSpec((1, 1, H), lambda b, be, nb: (be[b], 0, 0)),
                pl.BlockSpec((1, H, D), lambda b, be, nb: (be[b], 0, 0)),
                pl.BlockSpec((1, 1, D), lambda b, be, nb: (be[b], 0, 0)),
            ],
            out_specs=pl.BlockSpec((FB, D), lambda b, be, nb: (b, 0)),
        ),
        out_shape=jax.ShapeDtypeStruct((PADT, D), f32),
    )(be, nb, ordr, h2, W1, b1r, W2, b2r)

    mixed = pl.kernel(
        _sc_combine_body,
        compiler_params=sc_params,
        out_type=jax.ShapeDtypeStruct((N, D), f32),
        mesh=mesh,
        scratch_types=[
            pltpu.VMEM((N // (NC * NS),), i32),
            pltpu.VMEM((N // (NC * NS),), i32),
            pltpu.VMEM((N // (NC * NS),), f32),
            pltpu.VMEM((N // (NC * NS),), f32),
            pltpu.VMEM((N // (NC * NS), D), f32),
            pltpu.VMEM((N // (NC * NS), D), f32),
            pltpu.SemaphoreType.DMA,
        ],
    )(ys, pos1, pos2, p1f, p2f)

    TB = 512
    nt = N // TB
    out = pl.pallas_call(
        _final_kernel,
        grid=(nt,),
        in_specs=[
            pl.BlockSpec((TB, D), lambda t: (t, 0)),
            pl.BlockSpec((TB, D), lambda t: (t, 0)),
            pl.BlockSpec((D,), lambda t: (0,)),
            pl.BlockSpec((D,), lambda t: (0,)),
            pl.BlockSpec((D,), lambda t: (0,)),
            pl.BlockSpec((D,), lambda t: (0,)),
            pl.BlockSpec((D, C), lambda t: (0, 0)),
            pl.BlockSpec((C,), lambda t: (0,)),
        ],
        out_specs=pl.BlockSpec((TB, C), lambda t: (t, 0)),
        out_shape=jax.ShapeDtypeStruct((N, C), f32),
    )(h, mixed, g_moe, b_moe, g_out, b_out, Wc, bc)
    return out
